# baseline (speedup 1.0000x reference)
import functools

import jax
import jax.numpy as jnp
from jax import lax
from jax.experimental import pallas as pl
from jax.experimental.pallas import tpu as pltpu
from jax.experimental.pallas import tpu_sc as plsc

N = 10000
R = 20
H = 3
E = 320000
TN = 1000
NW = 32
EPW = E // NW
C = 16
NCH = EPW // C
NPS = 624
TAIL = N - 16 * NPS

_SC_PARAMS = pltpu.CompilerParams(use_tc_tiling_on_sc=False)


def _trans_body(h_ref, w_ref, alt_ref, art_ref, trans_ref, lg_ref):
    t = jnp.dot(h_ref[...], w_ref[0], preferred_element_type=jnp.float32)
    trans_ref[0] = t
    el = jnp.dot(t, alt_ref[...], preferred_element_type=jnp.float32)
    er = jnp.dot(t, art_ref[...], preferred_element_type=jnp.float32)
    lg_ref[0] = jnp.concatenate([el, er], axis=1)


def _trans_call(h, W_rel, a_l, a_r):
    in_dim = h.shape[1]
    ho = W_rel.shape[2]
    out = ho // H
    heads = jnp.arange(ho) // out
    offs = jnp.arange(ho) % out
    cols = jnp.arange(8)[None, :]
    alt = jnp.where(cols == heads[:, None], a_l[heads, offs][:, None], 0.0)
    art = jnp.where(cols == heads[:, None], a_r[heads, offs][:, None], 0.0)
    grid = (R, N // TN)
    return pl.pallas_call(
        _trans_body,
        grid=grid,
        in_specs=[
            pl.BlockSpec((TN, in_dim), lambda r, t: (t, 0)),
            pl.BlockSpec((1, in_dim, ho), lambda r, t: (r, 0, 0)),
            pl.BlockSpec((ho, 8), lambda r, t: (0, 0)),
            pl.BlockSpec((ho, 8), lambda r, t: (0, 0)),
        ],
        out_specs=[
            pl.BlockSpec((1, TN, ho), lambda r, t: (r, t, 0)),
            pl.BlockSpec((1, TN, 16), lambda r, t: (r, t, 0)),
        ],
        out_shape=[
            jax.ShapeDtypeStruct((R, N, ho), jnp.float32),
            jax.ShapeDtypeStruct((R, N, 16), jnp.float32),
        ],
    )(h, W_rel, alt, art)


def _vgather16(v, idx):
    dn = lax.GatherDimensionNumbers(
        offset_dims=(), collapsed_slice_dims=(0,), start_index_map=(0,))
    return lax.gather(v, idx[:, None], dn, (1,),
                      mode=lax.GatherScatterMode.PROMISE_IN_BOUNDS)


def _att_call(lg, rn_src, rn_dst, dst):
    zeros = jnp.zeros((N, 16), jnp.float32)
    mesh = plsc.VectorSubcoreMesh(core_axis_name="c", subcore_axis_name="s",
                                  num_cores=2, num_subcores=16)

    @functools.partial(
        pl.kernel,
        out_type=[jax.ShapeDtypeStruct((2, N, 16), jnp.float32),
                  jax.ShapeDtypeStruct((E, 16), jnp.float32)],
        mesh=mesh,
        compiler_params=_SC_PARAMS,
        scratch_types=[
            pltpu.VMEM_SHARED((N, 16), jnp.float32),
            pltpu.VMEM((C,), jnp.int32),
            pltpu.VMEM((C,), jnp.int32),
            pltpu.VMEM((C,), jnp.int32),
            pltpu.VMEM((C, 16), jnp.float32),
            pltpu.VMEM((C, 16), jnp.float32),
            pltpu.VMEM((C, 16), jnp.float32),
        ],
    )
    def k(lg_h, rns_h, rnd_h, dst_h, zero_h, den_out, ex_out,
          den_sh, rns_v, rnd_v, dst_v, lgs_v, lgd_v, ex_v):
        cc = lax.axis_index("c")
        ss = lax.axis_index("s")
        wid = cc * 16 + ss
        lane = lax.iota(jnp.int32, 16)
        shift_idx = jnp.where(lane < H, lane + 8, 0)
        pltpu.sync_copy(zero_h.at[pl.ds(ss * NPS, NPS)],
                        den_sh.at[pl.ds(ss * NPS, NPS)])
        @pl.when(ss == 15)
        def _zero_tail():
            pltpu.sync_copy(zero_h.at[pl.ds(16 * NPS, TAIL)],
                            den_sh.at[pl.ds(16 * NPS, TAIL)])
        plsc.subcore_barrier()

        def edge_body(i, carry):
            e = lgs_v[i] + _vgather16(lgd_v[i], shift_idx)
            e = jnp.where(e >= 0.0, e, 0.2 * e)
            ex_v[i] = jnp.where(lane < H, jnp.exp(e), 0.0)
            return carry

        def chunk_body(kk, carry):
            base = wid * EPW + kk * C
            pltpu.sync_copy(rns_h.at[pl.ds(base, C)], rns_v)
            pltpu.sync_copy(rnd_h.at[pl.ds(base, C)], rnd_v)
            pltpu.sync_copy(dst_h.at[pl.ds(base, C)], dst_v)
            pltpu.sync_copy(lg_h.at[rns_v], lgs_v)
            pltpu.sync_copy(lg_h.at[rnd_v], lgd_v)
            lax.fori_loop(0, C, edge_body, 0, unroll=2)
            pltpu.sync_copy(ex_v, ex_out.at[pl.ds(base, C)])
            pltpu.sync_copy(ex_v, den_sh.at[dst_v], add=True)
            return carry

        lax.fori_loop(0, NCH, chunk_body, 0)
        plsc.subcore_barrier()
        pltpu.sync_copy(den_sh.at[pl.ds(ss * NPS, NPS)],
                        den_out.at[cc, pl.ds(ss * NPS, NPS)])
        @pl.when(ss == 15)
        def _out_tail():
            pltpu.sync_copy(den_sh.at[pl.ds(16 * NPS, TAIL)],
                            den_out.at[cc, pl.ds(16 * NPS, TAIL)])

    return k(lg, rn_src, rn_dst, dst, zeros)


def _agg_call(trans, ex, rn_src, dst):
    ho = trans.shape[1]
    ng = ho // 16
    gph = ng // H
    zeros = jnp.zeros((N, ho), jnp.float32)
    mesh = plsc.VectorSubcoreMesh(core_axis_name="c", subcore_axis_name="s",
                                  num_cores=2, num_subcores=16)

    @functools.partial(
        pl.kernel,
        out_type=jax.ShapeDtypeStruct((2, N, ho), jnp.float32),
        mesh=mesh,
        compiler_params=_SC_PARAMS,
        scratch_types=[
            pltpu.VMEM_SHARED((N, ho), jnp.float32),
            pltpu.VMEM((C,), jnp.int32),
            pltpu.VMEM((C,), jnp.int32),
            pltpu.VMEM((C, 16), jnp.float32),
            pltpu.VMEM((C, ho), jnp.float32),
            pltpu.VMEM((C, ho), jnp.float32),
        ],
    )
    def k(trans_h, ex_h, rns_h, dst_h, zero_h, agg_out,
          acc_sh, rns_v, dst_v, ex_v, msg_v, sc_v):
        cc = lax.axis_index("c")
        ss = lax.axis_index("s")
        wid = cc * 16 + ss
        lane = lax.iota(jnp.int32, 16)
        pltpu.sync_copy(zero_h.at[pl.ds(ss * NPS, NPS)],
                        acc_sh.at[pl.ds(ss * NPS, NPS)])
        @pl.when(ss == 15)
        def _zero_tail():
            pltpu.sync_copy(zero_h.at[pl.ds(16 * NPS, TAIL)],
                            acc_sh.at[pl.ds(16 * NPS, TAIL)])
        plsc.subcore_barrier()

        def edge_body(i, carry):
            ex = ex_v[i]
            for hh in range(H):
                b = _vgather16(ex, lane * 0 + hh)
                for g in range(hh * gph, (hh + 1) * gph):
                    sc_v[i, pl.ds(g * 16, 16)] = msg_v[i, pl.ds(g * 16, 16)] * b
            return carry

        def chunk_body(kk, carry):
            base = wid * EPW + kk * C
            pltpu.sync_copy(rns_h.at[pl.ds(base, C)], rns_v)
            pltpu.sync_copy(dst_h.at[pl.ds(base, C)], dst_v)
            pltpu.sync_copy(ex_h.at[pl.ds(base, C)], ex_v)
            pltpu.sync_copy(trans_h.at[rns_v], msg_v)
            lax.fori_loop(0, C, edge_body, 0, unroll=2)
            pltpu.sync_copy(sc_v, acc_sh.at[dst_v], add=True)
            return carry

        lax.fori_loop(0, NCH, chunk_body, 0)
        plsc.subcore_barrier()
        pltpu.sync_copy(acc_sh.at[pl.ds(ss * NPS, NPS)],
                        agg_out.at[cc, pl.ds(ss * NPS, NPS)])
        @pl.when(ss == 15)
        def _out_tail():
            pltpu.sync_copy(acc_sh.at[pl.ds(16 * NPS, TAIL)],
                            agg_out.at[cc, pl.ds(16 * NPS, TAIL)])

    return k(trans, ex, rn_src, dst, zeros)


def _combine_body(h_ref, ws_ref, agg_ref, den_ref, exp_ref, o_ref, *, relu):
    s = jnp.dot(h_ref[...], ws_ref[...], preferred_element_type=jnp.float32)
    a = agg_ref[0] + agg_ref[1]
    d = den_ref[0] + den_ref[1]
    denf = jnp.dot(d, exp_ref[...], preferred_element_type=jnp.float32)
    o = a / (denf + 1e-9) + s
    o_ref[...] = jnp.maximum(o, 0.0) if relu else o


def _combine_call(h, W_self, agg2, den2, relu):
    in_dim = h.shape[1]
    ho = W_self.shape[1]
    out = ho // H
    expand = (jnp.arange(16)[:, None] == (jnp.arange(ho) // out)[None, :]).astype(jnp.float32)
    grid = (N // TN,)
    return pl.pallas_call(
        functools.partial(_combine_body, relu=relu),
        grid=grid,
        in_specs=[
            pl.BlockSpec((TN, in_dim), lambda t: (t, 0)),
            pl.BlockSpec((in_dim, ho), lambda t: (0, 0)),
            pl.BlockSpec((2, TN, ho), lambda t: (0, t, 0)),
            pl.BlockSpec((2, TN, 16), lambda t: (0, t, 0)),
            pl.BlockSpec((16, ho), lambda t: (0, 0)),
        ],
        out_specs=pl.BlockSpec((TN, ho), lambda t: (t, 0)),
        out_shape=jax.ShapeDtypeStruct((N, ho), jnp.float32),
    )(h, W_self, agg2, den2, expand)


def kernel(features, edge_index, edge_type, W_rel_0, a_l_0, a_r_0, W_self_0,
           W_rel_1, a_l_1, a_r_1, W_self_1, W_rel_2, a_l_2, a_r_2, W_self_2):
    src = edge_index[0]
    dst = edge_index[1]
    rn_src = edge_type * N + src
    rn_dst = edge_type * N + dst
    h = features
    layers = [
        (W_rel_0, a_l_0, a_r_0, W_self_0, True),
        (W_rel_1, a_l_1, a_r_1, W_self_1, True),
        (W_rel_2, a_l_2, a_r_2, W_self_2, False),
    ]
    for W_rel, a_l, a_r, W_self, relu in layers:
        trans, lg = _trans_call(h, W_rel, a_l, a_r)
        ho = W_rel.shape[2]
        den2, ex = _att_call(lg.reshape(R * N, 16), rn_src, rn_dst, dst)
        agg2 = _agg_call(trans.reshape(R * N, ho), ex, rn_src, dst)
        h = _combine_call(h, W_self, agg2, den2, relu)
    return h


# speedup vs baseline: 2.2848x; 2.2848x over previous
import functools

import jax
import jax.numpy as jnp
from jax import lax
from jax.experimental import pallas as pl
from jax.experimental.pallas import tpu as pltpu
from jax.experimental.pallas import tpu_sc as plsc

N = 10000
R = 20
H = 3
E = 320000
TN = 1000
NW = 32
EPW = E // NW
C = 80
NCH = EPW // C
NPS = 624
TAIL = N - 16 * NPS

_SC_PARAMS = pltpu.CompilerParams(use_tc_tiling_on_sc=False)
_MESH = dict(core_axis_name="c", subcore_axis_name="s",
             num_cores=2, num_subcores=16)


def _trans_body(h_ref, w_ref, alt_ref, art_ref, tlo_ref, thi_ref, lg_ref):
    t = jnp.dot(h_ref[...], w_ref[0], preferred_element_type=jnp.float32)
    hw = t.shape[1] // 2
    tlo_ref[0] = t[:, :hw]
    thi_ref[0] = t[:, hw:]
    el = jnp.dot(t, alt_ref[...], preferred_element_type=jnp.float32)
    er = jnp.dot(t, art_ref[...], preferred_element_type=jnp.float32)
    lg_ref[0] = jnp.concatenate([el, er], axis=1)


def _trans_call(h, W_rel, a_l, a_r):
    in_dim = h.shape[1]
    ho = W_rel.shape[2]
    hw = ho // 2
    out = ho // H
    heads = jnp.arange(ho) // out
    offs = jnp.arange(ho) % out
    cols = jnp.arange(8)[None, :]
    alt = jnp.where(cols == heads[:, None], a_l[heads, offs][:, None], 0.0)
    art = jnp.where(cols == heads[:, None], a_r[heads, offs][:, None], 0.0)
    grid = (R, N // TN)
    return pl.pallas_call(
        _trans_body,
        grid=grid,
        in_specs=[
            pl.BlockSpec((TN, in_dim), lambda r, t: (t, 0)),
            pl.BlockSpec((1, in_dim, ho), lambda r, t: (r, 0, 0)),
            pl.BlockSpec((ho, 8), lambda r, t: (0, 0)),
            pl.BlockSpec((ho, 8), lambda r, t: (0, 0)),
        ],
        out_specs=[
            pl.BlockSpec((1, TN, hw), lambda r, t: (r, t, 0)),
            pl.BlockSpec((1, TN, hw), lambda r, t: (r, t, 0)),
            pl.BlockSpec((1, TN, 16), lambda r, t: (r, t, 0)),
        ],
        out_shape=[
            jax.ShapeDtypeStruct((R, N, hw), jnp.float32),
            jax.ShapeDtypeStruct((R, N, hw), jnp.float32),
            jax.ShapeDtypeStruct((R, N, 16), jnp.float32),
        ],
    )(h, W_rel, alt, art)


def _vgather16(v, idx):
    dn = lax.GatherDimensionNumbers(
        offset_dims=(), collapsed_slice_dims=(0,), start_index_map=(0,))
    return lax.gather(v, idx[:, None], dn, (1,),
                      mode=lax.GatherScatterMode.PROMISE_IN_BOUNDS)


def _zero_shared(zero_h, sh, ss):
    pltpu.sync_copy(zero_h.at[pl.ds(ss * NPS, NPS)], sh.at[pl.ds(ss * NPS, NPS)])
    @pl.when(ss == 15)
    def _tail():
        pltpu.sync_copy(zero_h.at[pl.ds(16 * NPS, TAIL)],
                        sh.at[pl.ds(16 * NPS, TAIL)])


def _writeout_shared(sh, out_h, cc, ss):
    pltpu.sync_copy(sh.at[pl.ds(ss * NPS, NPS)],
                    out_h.at[cc, pl.ds(ss * NPS, NPS)])
    @pl.when(ss == 15)
    def _tail():
        pltpu.sync_copy(sh.at[pl.ds(16 * NPS, TAIL)],
                        out_h.at[cc, pl.ds(16 * NPS, TAIL)])


def _att_call(lg, rn_src, rn_dst, dst):
    zeros = jnp.zeros((N, 16), jnp.float32)
    mesh = plsc.VectorSubcoreMesh(**_MESH)

    @functools.partial(
        pl.kernel,
        out_type=[jax.ShapeDtypeStruct((2, N, 16), jnp.float32),
                  jax.ShapeDtypeStruct((E, 16), jnp.float32)],
        mesh=mesh,
        compiler_params=_SC_PARAMS,
        scratch_types=[
            pltpu.VMEM_SHARED((N, 16), jnp.float32),
            pltpu.VMEM((C,), jnp.int32),
            pltpu.VMEM((C,), jnp.int32),
            pltpu.VMEM((C,), jnp.int32),
            pltpu.VMEM((C, 16), jnp.float32),
            pltpu.VMEM((C, 16), jnp.float32),
            pltpu.VMEM((C, 16), jnp.float32),
            pltpu.SemaphoreType.DMA,
        ],
    )
    def k(lg_h, rns_h, rnd_h, dst_h, zero_h, den_out, ex_out,
          den_sh, rns_v, rnd_v, dst_v, lgs_v, lgd_v, ex_v, sem):
        cc = lax.axis_index("c")
        ss = lax.axis_index("s")
        wid = cc * 16 + ss
        lane = lax.iota(jnp.int32, 16)
        shift_idx = jnp.where(lane < H, lane + 8, 0)
        _zero_shared(zero_h, den_sh, ss)
        plsc.subcore_barrier()

        def edge_body(i, carry):
            e = lgs_v[i] + _vgather16(lgd_v[i], shift_idx)
            e = jnp.where(e >= 0.0, e, 0.2 * e)
            ex_v[i] = jnp.where(lane < H, jnp.exp(e), 0.0)
            return carry

        def chunk_body(kk, carry):
            base = wid * EPW + kk * C
            c1 = pltpu.async_copy(rns_h.at[pl.ds(base, C)], rns_v, sem)
            c2 = pltpu.async_copy(rnd_h.at[pl.ds(base, C)], rnd_v, sem)
            c3 = pltpu.async_copy(dst_h.at[pl.ds(base, C)], dst_v, sem)
            c1.wait(); c2.wait(); c3.wait()
            g1 = pltpu.async_copy(lg_h.at[rns_v], lgs_v, sem)
            g2 = pltpu.async_copy(lg_h.at[rnd_v], lgd_v, sem)
            g1.wait(); g2.wait()
            lax.fori_loop(0, C, edge_body, 0, unroll=2)
            pltpu.sync_copy(ex_v, ex_out.at[pl.ds(base, C)])
            pltpu.sync_copy(ex_v, den_sh.at[dst_v], add=True)
            return carry

        lax.fori_loop(0, NCH, chunk_body, 0)
        plsc.subcore_barrier()
        _writeout_shared(den_sh, den_out, cc, ss)

    return k(lg, rn_src, rn_dst, dst, zeros)


def _agg_call(trans_half, ex, rn_src, dst, goff):
    hw = trans_half.shape[1]
    ng = hw // 16
    gph = (2 * hw) // (16 * H)
    zeros = jnp.zeros((N, hw), jnp.float32)
    mesh = plsc.VectorSubcoreMesh(**_MESH)

    @functools.partial(
        pl.kernel,
        out_type=jax.ShapeDtypeStruct((2, N, hw), jnp.float32),
        mesh=mesh,
        compiler_params=_SC_PARAMS,
        scratch_types=[
            pltpu.VMEM_SHARED((N, hw), jnp.float32),
            pltpu.VMEM((C,), jnp.int32),
            pltpu.VMEM((C,), jnp.int32),
            pltpu.VMEM((C, 16), jnp.float32),
            pltpu.VMEM((C, hw), jnp.float32),
            pltpu.VMEM((C, hw), jnp.float32),
            pltpu.SemaphoreType.DMA,
        ],
    )
    def k(trans_h, ex_h, rns_h, dst_h, zero_h, agg_out,
          acc_sh, rns_v, dst_v, ex_v, msg_v, sc_v, sem):
        cc = lax.axis_index("c")
        ss = lax.axis_index("s")
        wid = cc * 16 + ss
        lane = lax.iota(jnp.int32, 16)
        _zero_shared(zero_h, acc_sh, ss)
        plsc.subcore_barrier()

        def edge_body(i, carry):
            ex = ex_v[i]
            b = [_vgather16(ex, lane * 0 + hh) for hh in range(H)]
            for g in range(ng):
                hh = (goff + g) // gph
                sc_v[i, pl.ds(g * 16, 16)] = msg_v[i, pl.ds(g * 16, 16)] * b[hh]
            return carry

        def chunk_body(kk, carry):
            base = wid * EPW + kk * C
            c1 = pltpu.async_copy(rns_h.at[pl.ds(base, C)], rns_v, sem)
            c2 = pltpu.async_copy(dst_h.at[pl.ds(base, C)], dst_v, sem)
            c3 = pltpu.async_copy(ex_h.at[pl.ds(base, C)], ex_v, sem)
            c1.wait(); c2.wait(); c3.wait()
            g1 = pltpu.async_copy(trans_h.at[rns_v], msg_v, sem)
            g1.wait()
            lax.fori_loop(0, C, edge_body, 0, unroll=2)
            pltpu.sync_copy(sc_v, acc_sh.at[dst_v], add=True)
            return carry

        lax.fori_loop(0, NCH, chunk_body, 0)
        plsc.subcore_barrier()
        _writeout_shared(acc_sh, agg_out, cc, ss)

    return k(trans_half, ex, rn_src, dst, zeros)


def _combine_body(h_ref, ws_ref, alo_ref, ahi_ref, den_ref, exp_ref, o_ref, *, relu):
    s = jnp.dot(h_ref[...], ws_ref[...], preferred_element_type=jnp.float32)
    a = jnp.concatenate([alo_ref[0] + alo_ref[1], ahi_ref[0] + ahi_ref[1]], axis=1)
    d = den_ref[0] + den_ref[1]
    denf = jnp.dot(d, exp_ref[...], preferred_element_type=jnp.float32)
    o = a / (denf + 1e-9) + s
    o_ref[...] = jnp.maximum(o, 0.0) if relu else o


def _combine_call(h, W_self, agg_lo, agg_hi, den2, relu):
    in_dim = h.shape[1]
    ho = W_self.shape[1]
    hw = ho // 2
    out = ho // H
    expand = (jnp.arange(16)[:, None] == (jnp.arange(ho) // out)[None, :]).astype(jnp.float32)
    grid = (N // TN,)
    return pl.pallas_call(
        functools.partial(_combine_body, relu=relu),
        grid=grid,
        in_specs=[
            pl.BlockSpec((TN, in_dim), lambda t: (t, 0)),
            pl.BlockSpec((in_dim, ho), lambda t: (0, 0)),
            pl.BlockSpec((2, TN, hw), lambda t: (0, t, 0)),
            pl.BlockSpec((2, TN, hw), lambda t: (0, t, 0)),
            pl.BlockSpec((2, TN, 16), lambda t: (0, t, 0)),
            pl.BlockSpec((16, ho), lambda t: (0, 0)),
        ],
        out_specs=pl.BlockSpec((TN, ho), lambda t: (t, 0)),
        out_shape=jax.ShapeDtypeStruct((N, ho), jnp.float32),
    )(h, W_self, agg_lo, agg_hi, den2, expand)


def kernel(features, edge_index, edge_type, W_rel_0, a_l_0, a_r_0, W_self_0,
           W_rel_1, a_l_1, a_r_1, W_self_1, W_rel_2, a_l_2, a_r_2, W_self_2):
    src = edge_index[0]
    dst = edge_index[1]
    rn_src = edge_type * N + src
    rn_dst = edge_type * N + dst
    h = features
    layers = [
        (W_rel_0, a_l_0, a_r_0, W_self_0, True),
        (W_rel_1, a_l_1, a_r_1, W_self_1, True),
        (W_rel_2, a_l_2, a_r_2, W_self_2, False),
    ]
    for W_rel, a_l, a_r, W_self, relu in layers:
        ho = W_rel.shape[2]
        hw = ho // 2
        tlo, thi, lg = _trans_call(h, W_rel, a_l, a_r)
        den2, ex = _att_call(lg.reshape(R * N, 16), rn_src, rn_dst, dst)
        agg_lo = _agg_call(tlo.reshape(R * N, hw), ex, rn_src, dst, 0)
        agg_hi = _agg_call(thi.reshape(R * N, hw), ex, rn_src, dst, hw // 16)
        h = _combine_call(h, W_self, agg_lo, agg_hi, den2, relu)
    return h


# speedup vs baseline: 2.4667x; 1.0796x over previous
import functools

import jax
import jax.numpy as jnp
from jax import lax
from jax.experimental import pallas as pl
from jax.experimental.pallas import tpu as pltpu
from jax.experimental.pallas import tpu_sc as plsc

N = 10000
R = 20
H = 3
E = 320000
TN = 1000
NW = 32
EPW = E // NW
CLO = 40
CHI = 80
NPS = 624
TAIL = N - 16 * NPS

_SC_PARAMS = pltpu.CompilerParams(use_tc_tiling_on_sc=False)
_MESH = dict(core_axis_name="c", subcore_axis_name="s",
             num_cores=2, num_subcores=16)


def _trans_body(h_ref, w_ref, alt_ref, art_ref, tlo_ref, thi_ref, lg_ref):
    t = jnp.dot(h_ref[...], w_ref[0], preferred_element_type=jnp.float32)
    hw = t.shape[1] // 2
    tlo_ref[0] = t[:, :hw]
    thi_ref[0] = t[:, hw:]
    el = jnp.dot(t, alt_ref[...], preferred_element_type=jnp.float32)
    er = jnp.dot(t, art_ref[...], preferred_element_type=jnp.float32)
    lg_ref[0] = jnp.concatenate([el, er], axis=1)


def _trans_call(h, W_rel, a_l, a_r):
    in_dim = h.shape[1]
    ho = W_rel.shape[2]
    hw = ho // 2
    out = ho // H
    heads = jnp.arange(ho) // out
    offs = jnp.arange(ho) % out
    cols = jnp.arange(8)[None, :]
    alt = jnp.where(cols == heads[:, None], a_l[heads, offs][:, None], 0.0)
    art = jnp.where(cols == heads[:, None], a_r[heads, offs][:, None], 0.0)
    grid = (R, N // TN)
    return pl.pallas_call(
        _trans_body,
        grid=grid,
        in_specs=[
            pl.BlockSpec((TN, in_dim), lambda r, t: (t, 0)),
            pl.BlockSpec((1, in_dim, ho), lambda r, t: (r, 0, 0)),
            pl.BlockSpec((ho, 8), lambda r, t: (0, 0)),
            pl.BlockSpec((ho, 8), lambda r, t: (0, 0)),
        ],
        out_specs=[
            pl.BlockSpec((1, TN, hw), lambda r, t: (r, t, 0)),
            pl.BlockSpec((1, TN, hw), lambda r, t: (r, t, 0)),
            pl.BlockSpec((1, TN, 16), lambda r, t: (r, t, 0)),
        ],
        out_shape=[
            jax.ShapeDtypeStruct((R, N, hw), jnp.float32),
            jax.ShapeDtypeStruct((R, N, hw), jnp.float32),
            jax.ShapeDtypeStruct((R, N, 16), jnp.float32),
        ],
    )(h, W_rel, alt, art)


def _vgather16(v, idx):
    dn = lax.GatherDimensionNumbers(
        offset_dims=(), collapsed_slice_dims=(0,), start_index_map=(0,))
    return lax.gather(v, idx[:, None], dn, (1,),
                      mode=lax.GatherScatterMode.PROMISE_IN_BOUNDS)


def _zero_shared(zero_h, sh, ss):
    pltpu.sync_copy(zero_h.at[pl.ds(ss * NPS, NPS)], sh.at[pl.ds(ss * NPS, NPS)])
    @pl.when(ss == 15)
    def _tail():
        pltpu.sync_copy(zero_h.at[pl.ds(16 * NPS, TAIL)],
                        sh.at[pl.ds(16 * NPS, TAIL)])


def _writeout_shared(sh, out_h, cc, ss):
    pltpu.sync_copy(sh.at[pl.ds(ss * NPS, NPS)],
                    out_h.at[cc, pl.ds(ss * NPS, NPS)])
    @pl.when(ss == 15)
    def _tail():
        pltpu.sync_copy(sh.at[pl.ds(16 * NPS, TAIL)],
                        out_h.at[cc, pl.ds(16 * NPS, TAIL)])


def _agg_lo_call(lg, trans_lo, rn_src, rn_dst, dst):
    hw = trans_lo.shape[1]
    ng = hw // 16
    gph = (2 * hw) // (16 * H)
    C = CLO
    NCH = EPW // C
    zeros16 = jnp.zeros((N, 16), jnp.float32)
    zeroshw = jnp.zeros((N, hw), jnp.float32)
    mesh = plsc.VectorSubcoreMesh(**_MESH)

    @functools.partial(
        pl.kernel,
        out_type=[jax.ShapeDtypeStruct((2, N, 16), jnp.float32),
                  jax.ShapeDtypeStruct((E, 16), jnp.float32),
                  jax.ShapeDtypeStruct((2, N, hw), jnp.float32)],
        mesh=mesh,
        compiler_params=_SC_PARAMS,
        scratch_types=[
            pltpu.VMEM_SHARED((N, 16), jnp.float32),
            pltpu.VMEM_SHARED((N, hw), jnp.float32),
            pltpu.VMEM((3, C), jnp.int32),
            pltpu.VMEM((3, C), jnp.int32),
            pltpu.VMEM((3, C), jnp.int32),
            pltpu.VMEM((2, C, 16), jnp.float32),
            pltpu.VMEM((2, C, 16), jnp.float32),
            pltpu.VMEM((2, C, hw), jnp.float32),
            pltpu.VMEM((2, C, 16), jnp.float32),
            pltpu.VMEM((2, C, hw), jnp.float32),
            pltpu.SemaphoreType.DMA((3,)),
            pltpu.SemaphoreType.DMA((2,)),
            pltpu.SemaphoreType.DMA((2,)),
        ],
    )
    def k(lg_h, tlo_h, rns_h, rnd_h, dst_h, z16_h, zhw_h,
          den_out, ex_out, agg_out,
          den_sh, acc_sh, rns_v, rnd_v, dst_v, lgs_v, lgd_v, msg_v, ex_v, sc_v,
          semL, semG, semS):
        cc = lax.axis_index("c")
        ss = lax.axis_index("s")
        wid = cc * 16 + ss
        base0 = wid * EPW
        lane = lax.iota(jnp.int32, 16)
        shift_idx = jnp.where(lane < H, lane + 8, 0)
        _zero_shared(z16_h, den_sh, ss)
        _zero_shared(zhw_h, acc_sh, ss)
        plsc.subcore_barrier()

        def fire_L(kk):
            s3 = kk % 3
            base = base0 + kk * C
            pltpu.async_copy(rns_h.at[pl.ds(base, C)], rns_v.at[s3], semL.at[s3])
            pltpu.async_copy(rnd_h.at[pl.ds(base, C)], rnd_v.at[s3], semL.at[s3])
            pltpu.async_copy(dst_h.at[pl.ds(base, C)], dst_v.at[s3], semL.at[s3])

        def wait_L(kk):
            s3 = kk % 3
            pltpu.make_async_copy(rns_h.at[pl.ds(0, C)], rns_v.at[s3], semL.at[s3]).wait()
            pltpu.make_async_copy(rnd_h.at[pl.ds(0, C)], rnd_v.at[s3], semL.at[s3]).wait()
            pltpu.make_async_copy(dst_h.at[pl.ds(0, C)], dst_v.at[s3], semL.at[s3]).wait()

        def fire_G(kk):
            s3 = kk % 3
            s2 = kk % 2
            pltpu.async_copy(lg_h.at[rns_v.at[s3]], lgs_v.at[s2], semG.at[s2])
            pltpu.async_copy(lg_h.at[rnd_v.at[s3]], lgd_v.at[s2], semG.at[s2])
            pltpu.async_copy(tlo_h.at[rns_v.at[s3]], msg_v.at[s2], semG.at[s2])

        def wait_G(kk):
            s3 = kk % 3
            s2 = kk % 2
            pltpu.make_async_copy(lg_h.at[rns_v.at[s3]], lgs_v.at[s2], semG.at[s2]).wait()
            pltpu.make_async_copy(lg_h.at[rnd_v.at[s3]], lgd_v.at[s2], semG.at[s2]).wait()
            pltpu.make_async_copy(tlo_h.at[rns_v.at[s3]], msg_v.at[s2], semG.at[s2]).wait()

        def fire_S(kk):
            s3 = kk % 3
            s2 = kk % 2
            base = base0 + kk * C
            pltpu.async_copy(ex_v.at[s2], ex_out.at[pl.ds(base, C)], semS.at[s2])
            pltpu.sync_copy(ex_v.at[s2], den_sh.at[dst_v.at[s3]], add=True)
            pltpu.sync_copy(sc_v.at[s2], acc_sh.at[dst_v.at[s3]], add=True)

        def wait_S(kk):
            s2 = kk % 2
            pltpu.make_async_copy(ex_v.at[s2], ex_out.at[pl.ds(0, C)], semS.at[s2]).wait()

        def compute(kk):
            s2 = kk % 2

            def edge_body(i, carry):
                e = lgs_v[s2, i] + _vgather16(lgd_v[s2, i], shift_idx)
                e = jnp.where(e >= 0.0, e, 0.2 * e)
                ex = jnp.where(lane < H, jnp.exp(e), 0.0)
                ex_v[s2, i] = ex
                b = [_vgather16(ex, lane * 0 + hh) for hh in range(H)]
                for g in range(ng):
                    hh = g // gph
                    sc_v[s2, i, pl.ds(g * 16, 16)] = (
                        msg_v[s2, i, pl.ds(g * 16, 16)] * b[hh])
                return carry

            lax.fori_loop(0, C, edge_body, 0, unroll=2)

        fire_L(0)
        wait_L(0)
        fire_G(0)
        fire_L(1)

        def chunk_body(kk, carry):
            wait_G(kk)
            @pl.when(kk >= 1)
            def _ws():
                wait_S(kk - 1)
            compute(kk)
            fire_S(kk)
            @pl.when(kk + 1 < NCH)
            def _next_g():
                wait_L(kk + 1)
                fire_G(kk + 1)
            @pl.when(kk + 2 < NCH)
            def _next_l():
                fire_L(kk + 2)
            return carry

        lax.fori_loop(0, NCH, chunk_body, 0)
        wait_S(NCH - 1)
        plsc.subcore_barrier()
        _writeout_shared(den_sh, den_out, cc, ss)
        _writeout_shared(acc_sh, agg_out, cc, ss)

    return k(lg, trans_lo, rn_src, rn_dst, dst, zeros16, zeroshw)


def _agg_hi_call(trans_hi, ex, rn_src, dst, goff):
    hw = trans_hi.shape[1]
    ng = hw // 16
    gph = (2 * hw) // (16 * H)
    C = CHI
    NCH = EPW // C
    zeros = jnp.zeros((N, hw), jnp.float32)
    mesh = plsc.VectorSubcoreMesh(**_MESH)

    @functools.partial(
        pl.kernel,
        out_type=jax.ShapeDtypeStruct((2, N, hw), jnp.float32),
        mesh=mesh,
        compiler_params=_SC_PARAMS,
        scratch_types=[
            pltpu.VMEM_SHARED((N, hw), jnp.float32),
            pltpu.VMEM((3, C), jnp.int32),
            pltpu.VMEM((3, C), jnp.int32),
            pltpu.VMEM((3, C, 16), jnp.float32),
            pltpu.VMEM((2, C, hw), jnp.float32),
            pltpu.VMEM((2, C, hw), jnp.float32),
            pltpu.SemaphoreType.DMA((3,)),
            pltpu.SemaphoreType.DMA((2,)),
            pltpu.SemaphoreType.DMA((2,)),
        ],
    )
    def k(thi_h, ex_h, rns_h, dst_h, zero_h, agg_out,
          acc_sh, rns_v, dst_v, ex_v, msg_v, sc_v, semL, semG, semS):
        cc = lax.axis_index("c")
        ss = lax.axis_index("s")
        wid = cc * 16 + ss
        base0 = wid * EPW
        lane = lax.iota(jnp.int32, 16)
        _zero_shared(zero_h, acc_sh, ss)
        plsc.subcore_barrier()

        def fire_L(kk):
            s3 = kk % 3
            base = base0 + kk * C
            pltpu.async_copy(rns_h.at[pl.ds(base, C)], rns_v.at[s3], semL.at[s3])
            pltpu.async_copy(dst_h.at[pl.ds(base, C)], dst_v.at[s3], semL.at[s3])
            pltpu.async_copy(ex_h.at[pl.ds(base, C)], ex_v.at[s3], semL.at[s3])

        def wait_L(kk):
            s3 = kk % 3
            pltpu.make_async_copy(rns_h.at[pl.ds(0, C)], rns_v.at[s3], semL.at[s3]).wait()
            pltpu.make_async_copy(dst_h.at[pl.ds(0, C)], dst_v.at[s3], semL.at[s3]).wait()
            pltpu.make_async_copy(ex_h.at[pl.ds(0, C)], ex_v.at[s3], semL.at[s3]).wait()

        def fire_G(kk):
            s3 = kk % 3
            s2 = kk % 2
            pltpu.async_copy(thi_h.at[rns_v.at[s3]], msg_v.at[s2], semG.at[s2])

        def wait_G(kk):
            s3 = kk % 3
            s2 = kk % 2
            pltpu.make_async_copy(thi_h.at[rns_v.at[s3]], msg_v.at[s2], semG.at[s2]).wait()

        def fire_S(kk):
            s3 = kk % 3
            s2 = kk % 2
            pltpu.sync_copy(sc_v.at[s2], acc_sh.at[dst_v.at[s3]], add=True)

        def wait_S(kk):
            del kk

        def compute(kk):
            s3 = kk % 3
            s2 = kk % 2

            def edge_body(i, carry):
                ex = ex_v[s3, i]
                b = [_vgather16(ex, lane * 0 + hh) for hh in range(H)]
                for g in range(ng):
                    hh = (goff + g) // gph
                    sc_v[s2, i, pl.ds(g * 16, 16)] = (
                        msg_v[s2, i, pl.ds(g * 16, 16)] * b[hh])
                return carry

            lax.fori_loop(0, C, edge_body, 0, unroll=2)

        fire_L(0)
        wait_L(0)
        fire_G(0)
        fire_L(1)

        def chunk_body(kk, carry):
            wait_G(kk)
            @pl.when(kk >= 1)
            def _ws():
                wait_S(kk - 1)
            compute(kk)
            fire_S(kk)
            @pl.when(kk + 1 < NCH)
            def _next_g():
                wait_L(kk + 1)
                fire_G(kk + 1)
            @pl.when(kk + 2 < NCH)
            def _next_l():
                fire_L(kk + 2)
            return carry

        lax.fori_loop(0, NCH, chunk_body, 0)
        wait_S(NCH - 1)
        plsc.subcore_barrier()
        _writeout_shared(acc_sh, agg_out, cc, ss)

    return k(trans_hi, ex, rn_src, dst, zeros)


def _combine_body(h_ref, ws_ref, alo_ref, ahi_ref, den_ref, exp_ref, o_ref, *, relu):
    s = jnp.dot(h_ref[...], ws_ref[...], preferred_element_type=jnp.float32)
    a = jnp.concatenate([alo_ref[0] + alo_ref[1], ahi_ref[0] + ahi_ref[1]], axis=1)
    d = den_ref[0] + den_ref[1]
    denf = jnp.dot(d, exp_ref[...], preferred_element_type=jnp.float32)
    o = a / (denf + 1e-9) + s
    o_ref[...] = jnp.maximum(o, 0.0) if relu else o


def _combine_call(h, W_self, agg_lo, agg_hi, den2, relu):
    in_dim = h.shape[1]
    ho = W_self.shape[1]
    hw = ho // 2
    out = ho // H
    expand = (jnp.arange(16)[:, None] == (jnp.arange(ho) // out)[None, :]).astype(jnp.float32)
    grid = (N // TN,)
    return pl.pallas_call(
        functools.partial(_combine_body, relu=relu),
        grid=grid,
        in_specs=[
            pl.BlockSpec((TN, in_dim), lambda t: (t, 0)),
            pl.BlockSpec((in_dim, ho), lambda t: (0, 0)),
            pl.BlockSpec((2, TN, hw), lambda t: (0, t, 0)),
            pl.BlockSpec((2, TN, hw), lambda t: (0, t, 0)),
            pl.BlockSpec((2, TN, 16), lambda t: (0, t, 0)),
            pl.BlockSpec((16, ho), lambda t: (0, 0)),
        ],
        out_specs=pl.BlockSpec((TN, ho), lambda t: (t, 0)),
        out_shape=jax.ShapeDtypeStruct((N, ho), jnp.float32),
    )(h, W_self, agg_lo, agg_hi, den2, expand)


def kernel(features, edge_index, edge_type, W_rel_0, a_l_0, a_r_0, W_self_0,
           W_rel_1, a_l_1, a_r_1, W_self_1, W_rel_2, a_l_2, a_r_2, W_self_2):
    src = edge_index[0]
    dst = edge_index[1]
    rn_src = edge_type * N + src
    rn_dst = edge_type * N + dst
    h = features
    layers = [
        (W_rel_0, a_l_0, a_r_0, W_self_0, True),
        (W_rel_1, a_l_1, a_r_1, W_self_1, True),
        (W_rel_2, a_l_2, a_r_2, W_self_2, False),
    ]
    for W_rel, a_l, a_r, W_self, relu in layers:
        ho = W_rel.shape[2]
        hw = ho // 2
        tlo, thi, lg = _trans_call(h, W_rel, a_l, a_r)
        den2, ex, agg_lo = _agg_lo_call(lg.reshape(R * N, 16),
                                        tlo.reshape(R * N, hw),
                                        rn_src, rn_dst, dst)
        agg_hi = _agg_hi_call(thi.reshape(R * N, hw), ex, rn_src, dst, hw // 16)
        h = _combine_call(h, W_self, agg_lo, agg_hi, den2, relu)
    return h


# speedup vs baseline: 2.6778x; 1.0856x over previous
import functools

import jax
import jax.numpy as jnp
from jax import lax
from jax.experimental import pallas as pl
from jax.experimental.pallas import tpu as pltpu
from jax.experimental.pallas import tpu_sc as plsc

N = 10000
R = 20
H = 3
E = 320000
TN = 1000
NW = 32
EPW = E // NW
C = 80
NCH = EPW // C
NPS = 624
TAIL = N - 16 * NPS

_SC_PARAMS = pltpu.CompilerParams(use_tc_tiling_on_sc=False)
_MESH = dict(core_axis_name="c", subcore_axis_name="s",
             num_cores=2, num_subcores=16)


def _trans_body(h_ref, w_ref, alt_ref, art_ref, tlo_ref, thi_ref, lg_ref):
    t = jnp.dot(h_ref[...], w_ref[0], preferred_element_type=jnp.float32)
    hw = t.shape[1] // 2
    tlo_ref[0] = t[:, :hw]
    thi_ref[0] = t[:, hw:]
    el = jnp.dot(t, alt_ref[...], preferred_element_type=jnp.float32)
    er = jnp.dot(t, art_ref[...], preferred_element_type=jnp.float32)
    lg_ref[0] = jnp.concatenate([el, er], axis=1)


def _trans_call(h, W_rel, a_l, a_r):
    in_dim = h.shape[1]
    ho = W_rel.shape[2]
    hw = ho // 2
    out = ho // H
    heads = jnp.arange(ho) // out
    offs = jnp.arange(ho) % out
    cols = jnp.arange(8)[None, :]
    alt = jnp.where(cols == heads[:, None], a_l[heads, offs][:, None], 0.0)
    art = jnp.where(cols == heads[:, None], a_r[heads, offs][:, None], 0.0)
    grid = (R, N // TN)
    return pl.pallas_call(
        _trans_body,
        grid=grid,
        in_specs=[
            pl.BlockSpec((TN, in_dim), lambda r, t: (t, 0)),
            pl.BlockSpec((1, in_dim, ho), lambda r, t: (r, 0, 0)),
            pl.BlockSpec((ho, 8), lambda r, t: (0, 0)),
            pl.BlockSpec((ho, 8), lambda r, t: (0, 0)),
        ],
        out_specs=[
            pl.BlockSpec((1, TN, hw), lambda r, t: (r, t, 0)),
            pl.BlockSpec((1, TN, hw), lambda r, t: (r, t, 0)),
            pl.BlockSpec((1, TN, 16), lambda r, t: (r, t, 0)),
        ],
        out_shape=[
            jax.ShapeDtypeStruct((R, N, hw), jnp.float32),
            jax.ShapeDtypeStruct((R, N, hw), jnp.float32),
            jax.ShapeDtypeStruct((R, N, 16), jnp.float32),
        ],
    )(h, W_rel, alt, art)


def _vgather16(v, idx):
    dn = lax.GatherDimensionNumbers(
        offset_dims=(), collapsed_slice_dims=(0,), start_index_map=(0,))
    return lax.gather(v, idx[:, None], dn, (1,),
                      mode=lax.GatherScatterMode.PROMISE_IN_BOUNDS)


def _zero_shared(zero_h, sh, ss):
    pltpu.sync_copy(zero_h.at[pl.ds(ss * NPS, NPS)], sh.at[pl.ds(ss * NPS, NPS)])
    @pl.when(ss == 15)
    def _tail():
        pltpu.sync_copy(zero_h.at[pl.ds(16 * NPS, TAIL)],
                        sh.at[pl.ds(16 * NPS, TAIL)])


def _writeout_shared(sh, out_h, cc, ss):
    pltpu.sync_copy(sh.at[pl.ds(ss * NPS, NPS)],
                    out_h.at[cc, pl.ds(ss * NPS, NPS)])
    @pl.when(ss == 15)
    def _tail():
        pltpu.sync_copy(sh.at[pl.ds(16 * NPS, TAIL)],
                        out_h.at[cc, pl.ds(16 * NPS, TAIL)])


def _edge_pass_call(lg, trans_half, rn_src, rn_dst, dst, goff, with_ex):
    hw = trans_half.shape[1]
    ng = hw // 16
    gph = (2 * hw) // (16 * H)
    W = hw + 16 if with_ex else hw
    zeros = jnp.zeros((N, W), jnp.float32)
    mesh = plsc.VectorSubcoreMesh(**_MESH)

    @functools.partial(
        pl.kernel,
        out_type=jax.ShapeDtypeStruct((2, N, W), jnp.float32),
        mesh=mesh,
        compiler_params=_SC_PARAMS,
        scratch_types=[
            pltpu.VMEM_SHARED((N, W), jnp.float32),
            pltpu.VMEM((3, C), jnp.int32),
            pltpu.VMEM((3, C), jnp.int32),
            pltpu.VMEM((3, C), jnp.int32),
            pltpu.VMEM((2, C, 16), jnp.float32),
            pltpu.VMEM((2, C, 16), jnp.float32),
            pltpu.VMEM((2, C, hw), jnp.float32),
            pltpu.VMEM((C, W), jnp.float32),
            pltpu.SemaphoreType.DMA((3,)),
            pltpu.SemaphoreType.DMA((2,)),
        ],
    )
    def k(lg_h, th_h, rns_h, rnd_h, dst_h, zero_h, agg_out,
          acc_sh, rns_v, rnd_v, dst_v, lgs_v, lgd_v, msg_v, sc_v, semL, semG):
        cc = lax.axis_index("c")
        ss = lax.axis_index("s")
        wid = cc * 16 + ss
        base0 = wid * EPW
        lane = lax.iota(jnp.int32, 16)
        shift_idx = jnp.where(lane < H, lane + 8, 0)
        _zero_shared(zero_h, acc_sh, ss)
        plsc.subcore_barrier()

        def fire_L(kk):
            s3 = kk % 3
            base = base0 + kk * C
            pltpu.async_copy(rns_h.at[pl.ds(base, C)], rns_v.at[s3], semL.at[s3])
            pltpu.async_copy(rnd_h.at[pl.ds(base, C)], rnd_v.at[s3], semL.at[s3])
            pltpu.async_copy(dst_h.at[pl.ds(base, C)], dst_v.at[s3], semL.at[s3])

        def wait_L(kk):
            s3 = kk % 3
            pltpu.make_async_copy(rns_h.at[pl.ds(0, C)], rns_v.at[s3], semL.at[s3]).wait()
            pltpu.make_async_copy(rnd_h.at[pl.ds(0, C)], rnd_v.at[s3], semL.at[s3]).wait()
            pltpu.make_async_copy(dst_h.at[pl.ds(0, C)], dst_v.at[s3], semL.at[s3]).wait()

        def fire_G(kk):
            s3 = kk % 3
            s2 = kk % 2
            pltpu.async_copy(lg_h.at[rns_v.at[s3]], lgs_v.at[s2], semG.at[s2])
            pltpu.async_copy(lg_h.at[rnd_v.at[s3]], lgd_v.at[s2], semG.at[s2])
            pltpu.async_copy(th_h.at[rns_v.at[s3]], msg_v.at[s2], semG.at[s2])

        def wait_G(kk):
            s3 = kk % 3
            s2 = kk % 2
            pltpu.make_async_copy(lg_h.at[rns_v.at[s3]], lgs_v.at[s2], semG.at[s2]).wait()
            pltpu.make_async_copy(lg_h.at[rnd_v.at[s3]], lgd_v.at[s2], semG.at[s2]).wait()
            pltpu.make_async_copy(th_h.at[rns_v.at[s3]], msg_v.at[s2], semG.at[s2]).wait()

        def compute(kk):
            s2 = kk % 2

            def edge_body(i, carry):
                e = lgs_v[s2, i] + _vgather16(lgd_v[s2, i], shift_idx)
                e = jnp.where(e >= 0.0, e, 0.2 * e)
                ex = jnp.where(lane < H, jnp.exp(e), 0.0)
                if with_ex:
                    sc_v[i, pl.ds(hw, 16)] = ex
                b = [_vgather16(ex, lane * 0 + hh) for hh in range(H)]
                for g in range(ng):
                    hh = (goff + g) // gph
                    sc_v[i, pl.ds(g * 16, 16)] = (
                        msg_v[s2, i, pl.ds(g * 16, 16)] * b[hh])
                return carry

            lax.fori_loop(0, C, edge_body, 0, unroll=2)

        fire_L(0)
        wait_L(0)
        fire_G(0)
        fire_L(1)

        def chunk_body(kk, carry):
            wait_G(kk)
            @pl.when(kk + 1 < NCH)
            def _next_g():
                wait_L(kk + 1)
                fire_G(kk + 1)
            compute(kk)
            pltpu.sync_copy(sc_v, acc_sh.at[dst_v.at[kk % 3]], add=True)
            @pl.when(kk + 2 < NCH)
            def _next_l():
                fire_L(kk + 2)
            return carry

        lax.fori_loop(0, NCH, chunk_body, 0)
        plsc.subcore_barrier()
        _writeout_shared(acc_sh, agg_out, cc, ss)

    return k(lg, trans_half, rn_src, rn_dst, dst, zeros)


def _combine_body(h_ref, ws_ref, alo_ref, ahi_ref, exp_ref, o_ref, *, relu, hw):
    s = jnp.dot(h_ref[...], ws_ref[...], preferred_element_type=jnp.float32)
    lo = alo_ref[0] + alo_ref[1]
    hi = ahi_ref[0] + ahi_ref[1]
    a = jnp.concatenate([lo[:, :hw], hi], axis=1)
    d = lo[:, hw:]
    denf = jnp.dot(d, exp_ref[...], preferred_element_type=jnp.float32)
    o = a / (denf + 1e-9) + s
    o_ref[...] = jnp.maximum(o, 0.0) if relu else o


def _combine_call(h, W_self, agg_lo, agg_hi, relu):
    in_dim = h.shape[1]
    ho = W_self.shape[1]
    hw = ho // 2
    out = ho // H
    expand = (jnp.arange(16)[:, None] == (jnp.arange(ho) // out)[None, :]).astype(jnp.float32)
    grid = (N // TN,)
    return pl.pallas_call(
        functools.partial(_combine_body, relu=relu, hw=hw),
        grid=grid,
        in_specs=[
            pl.BlockSpec((TN, in_dim), lambda t: (t, 0)),
            pl.BlockSpec((in_dim, ho), lambda t: (0, 0)),
            pl.BlockSpec((2, TN, hw + 16), lambda t: (0, t, 0)),
            pl.BlockSpec((2, TN, hw), lambda t: (0, t, 0)),
            pl.BlockSpec((16, ho), lambda t: (0, 0)),
        ],
        out_specs=pl.BlockSpec((TN, ho), lambda t: (t, 0)),
        out_shape=jax.ShapeDtypeStruct((N, ho), jnp.float32),
    )(h, W_self, agg_lo, agg_hi, expand)


def kernel(features, edge_index, edge_type, W_rel_0, a_l_0, a_r_0, W_self_0,
           W_rel_1, a_l_1, a_r_1, W_self_1, W_rel_2, a_l_2, a_r_2, W_self_2):
    src = edge_index[0]
    dst = edge_index[1]
    rn_src = edge_type * N + src
    rn_dst = edge_type * N + dst
    h = features
    layers = [
        (W_rel_0, a_l_0, a_r_0, W_self_0, True),
        (W_rel_1, a_l_1, a_r_1, W_self_1, True),
        (W_rel_2, a_l_2, a_r_2, W_self_2, False),
    ]
    for W_rel, a_l, a_r, W_self, relu in layers:
        ho = W_rel.shape[2]
        hw = ho // 2
        tlo, thi, lg = _trans_call(h, W_rel, a_l, a_r)
        lgf = lg.reshape(R * N, 16)
        agg_lo = _edge_pass_call(lgf, tlo.reshape(R * N, hw),
                                 rn_src, rn_dst, dst, 0, True)
        agg_hi = _edge_pass_call(lgf, thi.reshape(R * N, hw),
                                 rn_src, rn_dst, dst, hw // 16, False)
        h = _combine_call(h, W_self, agg_lo, agg_hi, relu)
    return h


# speedup vs baseline: 2.6802x; 1.0009x over previous
import functools

import jax
import jax.numpy as jnp
from jax import lax
from jax.experimental import pallas as pl
from jax.experimental.pallas import tpu as pltpu
from jax.experimental.pallas import tpu_sc as plsc

N = 10000
R = 20
H = 3
E = 320000
TN = 1000
NW = 32
EPW = E // NW
C = 80
NCH = EPW // C
NPS = 624
TAIL = N - 16 * NPS

_SC_PARAMS = pltpu.CompilerParams(use_tc_tiling_on_sc=False)
_MESH = dict(core_axis_name="c", subcore_axis_name="s",
             num_cores=2, num_subcores=16)


def _trans_body(h_ref, w_ref, alt_ref, art_ref, tlo_ref, thi_ref, lg_ref):
    t = jnp.dot(h_ref[...], w_ref[0], preferred_element_type=jnp.float32)
    hw = t.shape[1] // 2
    tlo_ref[0] = t[:, :hw]
    thi_ref[0] = t[:, hw:]
    el = jnp.dot(t, alt_ref[...], preferred_element_type=jnp.float32)
    er = jnp.dot(t, art_ref[...], preferred_element_type=jnp.float32)
    lg_ref[0] = jnp.concatenate([el, er], axis=1)


def _trans_call(h, W_rel, a_l, a_r):
    in_dim = h.shape[1]
    ho = W_rel.shape[2]
    hw = ho // 2
    out = ho // H
    heads = jnp.arange(ho) // out
    offs = jnp.arange(ho) % out
    cols = jnp.arange(8)[None, :]
    alt = jnp.where(cols == heads[:, None], a_l[heads, offs][:, None], 0.0)
    art = jnp.where(cols == heads[:, None], a_r[heads, offs][:, None], 0.0)
    grid = (R, N // TN)
    return pl.pallas_call(
        _trans_body,
        grid=grid,
        in_specs=[
            pl.BlockSpec((TN, in_dim), lambda r, t: (t, 0)),
            pl.BlockSpec((1, in_dim, ho), lambda r, t: (r, 0, 0)),
            pl.BlockSpec((ho, 8), lambda r, t: (0, 0)),
            pl.BlockSpec((ho, 8), lambda r, t: (0, 0)),
        ],
        out_specs=[
            pl.BlockSpec((1, TN, hw), lambda r, t: (r, t, 0)),
            pl.BlockSpec((1, TN, hw), lambda r, t: (r, t, 0)),
            pl.BlockSpec((1, TN, 16), lambda r, t: (r, t, 0)),
        ],
        out_shape=[
            jax.ShapeDtypeStruct((R, N, hw), jnp.float32),
            jax.ShapeDtypeStruct((R, N, hw), jnp.float32),
            jax.ShapeDtypeStruct((R, N, 16), jnp.float32),
        ],
    )(h, W_rel, alt, art)


def _vgather16(v, idx):
    dn = lax.GatherDimensionNumbers(
        offset_dims=(), collapsed_slice_dims=(0,), start_index_map=(0,))
    return lax.gather(v, idx[:, None], dn, (1,),
                      mode=lax.GatherScatterMode.PROMISE_IN_BOUNDS)


def _zero_shared(zero_h, sh, ss):
    pltpu.sync_copy(zero_h.at[pl.ds(ss * NPS, NPS)], sh.at[pl.ds(ss * NPS, NPS)])
    @pl.when(ss == 15)
    def _tail():
        pltpu.sync_copy(zero_h.at[pl.ds(16 * NPS, TAIL)],
                        sh.at[pl.ds(16 * NPS, TAIL)])


def _writeout_shared(sh, out_h, cc, ss):
    pltpu.sync_copy(sh.at[pl.ds(ss * NPS, NPS)],
                    out_h.at[cc, pl.ds(ss * NPS, NPS)])
    @pl.when(ss == 15)
    def _tail():
        pltpu.sync_copy(sh.at[pl.ds(16 * NPS, TAIL)],
                        out_h.at[cc, pl.ds(16 * NPS, TAIL)])


def _edge_pass_call(lg, trans_half, rn_src, rn_dst, dst, goff, with_ex):
    hw = trans_half.shape[1]
    ng = hw // 16
    gph = (2 * hw) // (16 * H)
    W = hw + 16 if with_ex else hw
    zeros = jnp.zeros((N, W), jnp.float32)
    mesh = plsc.VectorSubcoreMesh(**_MESH)

    @functools.partial(
        pl.kernel,
        out_type=jax.ShapeDtypeStruct((2, N, W), jnp.float32),
        mesh=mesh,
        compiler_params=_SC_PARAMS,
        scratch_types=[
            pltpu.VMEM_SHARED((N, W), jnp.float32),
            pltpu.VMEM((3, C), jnp.int32),
            pltpu.VMEM((3, C), jnp.int32),
            pltpu.VMEM((3, C), jnp.int32),
            pltpu.VMEM((2, C, 16), jnp.float32),
            pltpu.VMEM((2, C, 16), jnp.float32),
            pltpu.VMEM((2, C, hw), jnp.float32),
            pltpu.VMEM((C, W), jnp.float32),
            pltpu.SemaphoreType.DMA((3,)),
            pltpu.SemaphoreType.DMA((2,)),
        ],
    )
    def k(lg_h, th_h, rns_h, rnd_h, dst_h, zero_h, agg_out,
          acc_sh, rns_v, rnd_v, dst_v, lgs_v, lgd_v, msg_v, sc_v, semL, semG):
        cc = lax.axis_index("c")
        ss = lax.axis_index("s")
        wid = cc * 16 + ss
        base0 = wid * EPW
        lane = lax.iota(jnp.int32, 16)
        shift_idx = jnp.where(lane < H, lane + 8, 0)
        _zero_shared(zero_h, acc_sh, ss)
        plsc.subcore_barrier()

        def fire_L(kk):
            s3 = kk % 3
            base = base0 + kk * C
            pltpu.async_copy(rns_h.at[pl.ds(base, C)], rns_v.at[s3], semL.at[s3])
            pltpu.async_copy(rnd_h.at[pl.ds(base, C)], rnd_v.at[s3], semL.at[s3])
            pltpu.async_copy(dst_h.at[pl.ds(base, C)], dst_v.at[s3], semL.at[s3])

        def wait_L(kk):
            s3 = kk % 3
            pltpu.make_async_copy(rns_h.at[pl.ds(0, C)], rns_v.at[s3], semL.at[s3]).wait()
            pltpu.make_async_copy(rnd_h.at[pl.ds(0, C)], rnd_v.at[s3], semL.at[s3]).wait()
            pltpu.make_async_copy(dst_h.at[pl.ds(0, C)], dst_v.at[s3], semL.at[s3]).wait()

        def fire_G(kk):
            s3 = kk % 3
            s2 = kk % 2
            pltpu.async_copy(lg_h.at[rns_v.at[s3]], lgs_v.at[s2], semG.at[s2])
            pltpu.async_copy(lg_h.at[rnd_v.at[s3]], lgd_v.at[s2], semG.at[s2])
            pltpu.async_copy(th_h.at[rns_v.at[s3]], msg_v.at[s2], semG.at[s2])

        def wait_G(kk):
            s3 = kk % 3
            s2 = kk % 2
            pltpu.make_async_copy(lg_h.at[rns_v.at[s3]], lgs_v.at[s2], semG.at[s2]).wait()
            pltpu.make_async_copy(lg_h.at[rnd_v.at[s3]], lgd_v.at[s2], semG.at[s2]).wait()
            pltpu.make_async_copy(th_h.at[rns_v.at[s3]], msg_v.at[s2], semG.at[s2]).wait()

        def compute(kk):
            s2 = kk % 2

            def edge_body(i, carry):
                e = lgs_v[s2, i] + _vgather16(lgd_v[s2, i], shift_idx)
                e = jnp.where(e >= 0.0, e, 0.2 * e)
                ex = jnp.where(lane < H, jnp.exp(e), 0.0)
                if with_ex:
                    sc_v[i, pl.ds(hw, 16)] = ex
                b = [_vgather16(ex, lane * 0 + hh) for hh in range(H)]
                for g in range(ng):
                    hh = (goff + g) // gph
                    sc_v[i, pl.ds(g * 16, 16)] = (
                        msg_v[s2, i, pl.ds(g * 16, 16)] * b[hh])
                return carry

            lax.fori_loop(0, C, edge_body, 0, unroll=4)

        fire_L(0)
        wait_L(0)
        fire_G(0)
        fire_L(1)

        def chunk_body(kk, carry):
            wait_G(kk)
            @pl.when(kk + 1 < NCH)
            def _next_g():
                wait_L(kk + 1)
                fire_G(kk + 1)
            compute(kk)
            pltpu.sync_copy(sc_v, acc_sh.at[dst_v.at[kk % 3]], add=True)
            @pl.when(kk + 2 < NCH)
            def _next_l():
                fire_L(kk + 2)
            return carry

        lax.fori_loop(0, NCH, chunk_body, 0)
        plsc.subcore_barrier()
        _writeout_shared(acc_sh, agg_out, cc, ss)

    return k(lg, trans_half, rn_src, rn_dst, dst, zeros)


def _combine_body(h_ref, ws_ref, alo_ref, ahi_ref, exp_ref, o_ref, *, relu, hw):
    s = jnp.dot(h_ref[...], ws_ref[...], preferred_element_type=jnp.float32)
    lo = alo_ref[0] + alo_ref[1]
    hi = ahi_ref[0] + ahi_ref[1]
    a = jnp.concatenate([lo[:, :hw], hi], axis=1)
    d = lo[:, hw:]
    denf = jnp.dot(d, exp_ref[...], preferred_element_type=jnp.float32)
    o = a / (denf + 1e-9) + s
    o_ref[...] = jnp.maximum(o, 0.0) if relu else o


def _combine_call(h, W_self, agg_lo, agg_hi, relu):
    in_dim = h.shape[1]
    ho = W_self.shape[1]
    hw = ho // 2
    out = ho // H
    expand = (jnp.arange(16)[:, None] == (jnp.arange(ho) // out)[None, :]).astype(jnp.float32)
    grid = (N // TN,)
    return pl.pallas_call(
        functools.partial(_combine_body, relu=relu, hw=hw),
        grid=grid,
        in_specs=[
            pl.BlockSpec((TN, in_dim), lambda t: (t, 0)),
            pl.BlockSpec((in_dim, ho), lambda t: (0, 0)),
            pl.BlockSpec((2, TN, hw + 16), lambda t: (0, t, 0)),
            pl.BlockSpec((2, TN, hw), lambda t: (0, t, 0)),
            pl.BlockSpec((16, ho), lambda t: (0, 0)),
        ],
        out_specs=pl.BlockSpec((TN, ho), lambda t: (t, 0)),
        out_shape=jax.ShapeDtypeStruct((N, ho), jnp.float32),
    )(h, W_self, agg_lo, agg_hi, expand)


def kernel(features, edge_index, edge_type, W_rel_0, a_l_0, a_r_0, W_self_0,
           W_rel_1, a_l_1, a_r_1, W_self_1, W_rel_2, a_l_2, a_r_2, W_self_2):
    src = edge_index[0]
    dst = edge_index[1]
    rn_src = edge_type * N + src
    rn_dst = edge_type * N + dst
    h = features
    layers = [
        (W_rel_0, a_l_0, a_r_0, W_self_0, True),
        (W_rel_1, a_l_1, a_r_1, W_self_1, True),
        (W_rel_2, a_l_2, a_r_2, W_self_2, False),
    ]
    for W_rel, a_l, a_r, W_self, relu in layers:
        ho = W_rel.shape[2]
        hw = ho // 2
        tlo, thi, lg = _trans_call(h, W_rel, a_l, a_r)
        lgf = lg.reshape(R * N, 16)
        agg_lo = _edge_pass_call(lgf, tlo.reshape(R * N, hw),
                                 rn_src, rn_dst, dst, 0, True)
        agg_hi = _edge_pass_call(lgf, thi.reshape(R * N, hw),
                                 rn_src, rn_dst, dst, hw // 16, False)
        h = _combine_call(h, W_self, agg_lo, agg_hi, relu)
    return h
